# Initial kernel scaffold; baseline (speedup 1.0000x reference)
import jax
import jax.numpy as jnp
from jax.experimental import pallas as pl


def kernel(x, edge_index, W):
    raise NotImplementedError("write your pallas kernel here")


# speedup vs baseline: 26.9892x; 26.9892x over previous
import jax
import jax.numpy as jnp
from jax import lax
from jax.experimental import pallas as pl
from jax.experimental.pallas import tpu as pltpu
from jax.experimental.pallas import tpu_sc as plsc

N_NODES = 10000
D_IN = 128
D_OUT = 128
N_EDGES = 320000

NC, NS = 2, 16
EDGES_PER_TILE = N_EDGES // (NC * NS)
EDGE_K = 80
SB = 25
NB = EDGES_PER_TILE // (EDGE_K * SB)
N_PAD = 10240
ROWS_PER_TILE = N_PAD // NS
HR, HC = N_PAD // 128, 128

_MESH = plsc.VectorSubcoreMesh(core_axis_name="c", subcore_axis_name="s")


def _hist_body(rows_hbm, cnt_hbm, idx_v, hist_v):
    c = lax.axis_index("c")
    s = lax.axis_index("s")

    @pl.loop(0, HR)
    def _(r):
        for l in range(HC // 16):
            hist_v[r, pl.ds(l * 16, 16)] = jnp.zeros((16,), jnp.int32)

    @pl.loop(0, NB)
    def _(b):
        pltpu.sync_copy(rows_hbm.at[c, s, b], idx_v)

        @pl.loop(0, SB)
        def _(t):
            for l in range(EDGE_K // 16):
                v = idx_v[t, pl.ds(l * 16, 16)]
                cnts, last = plsc.scan_count(v)
                hi = lax.shift_right_logical(v, 7)
                lo = lax.bitwise_and(v, 127)
                plsc.addupdate_scatter(hist_v, [hi, lo], cnts, mask=last)

    pltpu.sync_copy(hist_v, cnt_hbm.at[c, s])


_hist = pl.kernel(
    _hist_body,
    out_type=jax.ShapeDtypeStruct((NC, NS, HR, HC), jnp.int32),
    mesh=_MESH,
    scratch_types=[
        pltpu.VMEM((SB, EDGE_K), jnp.int32),
        pltpu.VMEM((HR, HC), jnp.int32),
    ],
    compiler_params=pltpu.CompilerParams(needs_layout_passes=False),
)


def _agg_body(z_hbm, rows_hbm, cols_hbm, zeros_hbm, out_hbm,
              ridx_v, cidx_v, upd0, upd1, acc_sp, gsem0, gsem1):
    c = lax.axis_index("c")
    s = lax.axis_index("s")
    base = s * ROWS_PER_TILE
    pltpu.sync_copy(zeros_hbm, acc_sp.at[pl.ds(base, ROWS_PER_TILE)])
    plsc.subcore_barrier()

    def gather(t, buf, sem):
        pltpu.async_copy(z_hbm.at[cidx_v.at[t]], buf, sem).wait()

    def scatter(t, buf):
        pltpu.sync_copy(buf, acc_sp.at[ridx_v.at[t]], add=True)

    @pl.loop(0, NB)
    def _(b):
        pltpu.sync_copy(rows_hbm.at[c, s, b], ridx_v)
        pltpu.sync_copy(cols_hbm.at[c, s, b], cidx_v)
        for t in range(SB):
            buf, sem = (upd0, gsem0) if t % 2 == 0 else (upd1, gsem1)
            gather(t, buf, sem)
            scatter(t, buf)

    plsc.subcore_barrier()
    pltpu.sync_copy(acc_sp.at[pl.ds(base, ROWS_PER_TILE)],
                    out_hbm.at[c, pl.ds(base, ROWS_PER_TILE)])


_agg = pl.kernel(
    _agg_body,
    out_type=jax.ShapeDtypeStruct((NC, N_PAD, D_OUT), jnp.float32),
    mesh=_MESH,
    scratch_types=[
        pltpu.VMEM((SB, EDGE_K), jnp.int32),
        pltpu.VMEM((SB, EDGE_K), jnp.int32),
        pltpu.VMEM((EDGE_K, D_OUT), jnp.float32),
        pltpu.VMEM((EDGE_K, D_OUT), jnp.float32),
        pltpu.VMEM_SHARED((N_PAD, D_OUT), jnp.float32),
        pltpu.SemaphoreType.DMA,
        pltpu.SemaphoreType.DMA,
    ],
)


_MM_BLOCK = 2000


def _mm_body(x_ref, w_ref, o_ref):
    o_ref[...] = jnp.dot(x_ref[...], w_ref[...],
                         preferred_element_type=jnp.float32)


def _matmul(x, w):
    return pl.pallas_call(
        _mm_body,
        grid=(N_NODES // _MM_BLOCK,),
        in_specs=[pl.BlockSpec((_MM_BLOCK, D_IN), lambda i: (i, 0)),
                  pl.BlockSpec((D_IN, D_OUT), lambda i: (0, 0))],
        out_specs=pl.BlockSpec((_MM_BLOCK, D_OUT), lambda i: (i, 0)),
        out_shape=jax.ShapeDtypeStruct((N_NODES, D_OUT), jnp.float32),
    )(x, w)


def _cntsum_body(c_ref, o_ref):
    o_ref[...] = jnp.sum(c_ref[...], axis=(0, 1)).astype(jnp.float32)


def _cntsum(cnt4):
    return pl.pallas_call(
        _cntsum_body,
        out_shape=jax.ShapeDtypeStruct((HR, HC), jnp.float32),
    )(cnt4)


_EW_BLOCK = 2000


def _scale_body(c_ref, x_ref, o_ref):
    deg = 2.0 + c_ref[...]
    o_ref[...] = x_ref[...] * lax.rsqrt(deg)


def _scale(cnt_col, xw):
    return pl.pallas_call(
        _scale_body,
        grid=(N_NODES // _EW_BLOCK,),
        in_specs=[pl.BlockSpec((_EW_BLOCK, 1), lambda i: (i, 0)),
                  pl.BlockSpec((_EW_BLOCK, D_OUT), lambda i: (i, 0))],
        out_specs=pl.BlockSpec((_EW_BLOCK, D_OUT), lambda i: (i, 0)),
        out_shape=jax.ShapeDtypeStruct((N_NODES, D_OUT), jnp.float32),
    )(cnt_col, xw)


def _final_body(a_ref, c_ref, x_ref, o_ref):
    deg = 2.0 + c_ref[...]
    dinv = lax.rsqrt(deg)
    tot = a_ref[0] + a_ref[1]
    o_ref[...] = jnp.maximum(dinv * tot + (2.0 / deg) * x_ref[...], 0.0)


def _final(acc, cnt_col, xw):
    return pl.pallas_call(
        _final_body,
        grid=(N_NODES // _EW_BLOCK,),
        in_specs=[pl.BlockSpec((NC, _EW_BLOCK, D_OUT), lambda i: (0, i, 0)),
                  pl.BlockSpec((_EW_BLOCK, 1), lambda i: (i, 0)),
                  pl.BlockSpec((_EW_BLOCK, D_OUT), lambda i: (i, 0))],
        out_specs=pl.BlockSpec((_EW_BLOCK, D_OUT), lambda i: (i, 0)),
        out_shape=jax.ShapeDtypeStruct((N_NODES, D_OUT), jnp.float32),
    )(acc, cnt_col, xw)


def kernel(x, edge_index, W):
    rows = edge_index[0].reshape(NC, NS, NB, SB, EDGE_K)
    cols = edge_index[1].reshape(NC, NS, NB, SB, EDGE_K)
    zeros128 = jnp.zeros((ROWS_PER_TILE, D_OUT), jnp.float32)

    cnt4 = _hist(rows)
    cntf = _cntsum(cnt4)
    cnt_col = cntf.reshape(N_PAD)[:N_NODES].reshape(N_NODES, 1)
    xw = _matmul(x, W)
    z = _scale(cnt_col, xw)
    acc = _agg(z, rows, cols, zeros128)
    return _final(acc, cnt_col, xw)


# speedup vs baseline: 37.9398x; 1.4057x over previous
import jax
import jax.numpy as jnp
from jax import lax
from jax.experimental import pallas as pl
from jax.experimental.pallas import tpu as pltpu
from jax.experimental.pallas import tpu_sc as plsc

N_NODES = 10000
D_IN = 128
D_OUT = 128
N_EDGES = 320000

NC, NS = 2, 16
EDGES_PER_TILE = N_EDGES // (NC * NS)
EDGE_K = 80
SB = 25
NB = EDGES_PER_TILE // (EDGE_K * SB)
N_PAD = 10240
ROWS_PER_TILE = N_PAD // NS
HR, HC = N_PAD // 128, 128

_MESH = plsc.VectorSubcoreMesh(core_axis_name="c", subcore_axis_name="s")


def _hist_body(rows_hbm, cnt_hbm, idx_v, hist_v):
    c = lax.axis_index("c")
    s = lax.axis_index("s")

    @pl.loop(0, HR)
    def _(r):
        for l in range(HC // 16):
            hist_v[r, pl.ds(l * 16, 16)] = jnp.zeros((16,), jnp.int32)

    @pl.loop(0, NB)
    def _(b):
        pltpu.sync_copy(rows_hbm.at[c, s, b], idx_v)

        @pl.loop(0, SB)
        def _(t):
            for l in range(EDGE_K // 16):
                v = idx_v[t, pl.ds(l * 16, 16)]
                cnts, last = plsc.scan_count(v)
                hi = lax.shift_right_logical(v, 7)
                lo = lax.bitwise_and(v, 127)
                plsc.addupdate_scatter(hist_v, [hi, lo], cnts, mask=last)

    pltpu.sync_copy(hist_v, cnt_hbm.at[c, s])


_hist = pl.kernel(
    _hist_body,
    out_type=jax.ShapeDtypeStruct((NC, NS, HR, HC), jnp.int32),
    mesh=_MESH,
    scratch_types=[
        pltpu.VMEM((SB, EDGE_K), jnp.int32),
        pltpu.VMEM((HR, HC), jnp.int32),
    ],
    compiler_params=pltpu.CompilerParams(needs_layout_passes=False),
)


def _agg_body(z_hbm, rows_hbm, cols_hbm, zeros_hbm, out_hbm,
              ridx_v, cidx_v, upd0, upd1, acc_sp, gsem0, gsem1, ssem0, ssem1):
    c = lax.axis_index("c")
    s = lax.axis_index("s")
    base = s * ROWS_PER_TILE
    pltpu.sync_copy(zeros_hbm, acc_sp.at[pl.ds(base, ROWS_PER_TILE)])
    plsc.subcore_barrier()

    bufs = (upd0, upd1)
    gsems = (gsem0, gsem1)
    ssems = (ssem0, ssem1)

    @pl.loop(0, NB)
    def _(b):
        pltpu.sync_copy(rows_hbm.at[c, s, b], ridx_v)
        pltpu.sync_copy(cols_hbm.at[c, s, b], cidx_v)
        gd = [None] * SB
        sd = [None] * SB
        gd[0] = pltpu.async_copy(z_hbm.at[cidx_v.at[0]], upd0, gsem0)
        gd[1] = pltpu.async_copy(z_hbm.at[cidx_v.at[1]], upd1, gsem1)
        for t in range(SB):
            p = t % 2
            gd[t].wait()
            sd[t] = pltpu.async_copy(bufs[p], acc_sp.at[ridx_v.at[t]],
                                     ssems[p], add=True)
            if t + 2 < SB:
                sd[t].wait()
                gd[t + 2] = pltpu.async_copy(z_hbm.at[cidx_v.at[t + 2]],
                                             bufs[p], gsems[p])
        sd[SB - 2].wait()
        sd[SB - 1].wait()

    plsc.subcore_barrier()
    pltpu.sync_copy(acc_sp.at[pl.ds(base, ROWS_PER_TILE)],
                    out_hbm.at[c, pl.ds(base, ROWS_PER_TILE)])


_agg = pl.kernel(
    _agg_body,
    out_type=jax.ShapeDtypeStruct((NC, N_PAD, D_OUT), jnp.float32),
    mesh=_MESH,
    scratch_types=[
        pltpu.VMEM((SB, EDGE_K), jnp.int32),
        pltpu.VMEM((SB, EDGE_K), jnp.int32),
        pltpu.VMEM((EDGE_K, D_OUT), jnp.float32),
        pltpu.VMEM((EDGE_K, D_OUT), jnp.float32),
        pltpu.VMEM_SHARED((N_PAD, D_OUT), jnp.float32),
        pltpu.SemaphoreType.DMA,
        pltpu.SemaphoreType.DMA,
        pltpu.SemaphoreType.DMA,
        pltpu.SemaphoreType.DMA,
    ],
)


_MM_BLOCK = 2000


def _mm_body(x_ref, w_ref, o_ref):
    o_ref[...] = jnp.dot(x_ref[...], w_ref[...],
                         preferred_element_type=jnp.float32)


def _matmul(x, w):
    return pl.pallas_call(
        _mm_body,
        grid=(N_NODES // _MM_BLOCK,),
        in_specs=[pl.BlockSpec((_MM_BLOCK, D_IN), lambda i: (i, 0)),
                  pl.BlockSpec((D_IN, D_OUT), lambda i: (0, 0))],
        out_specs=pl.BlockSpec((_MM_BLOCK, D_OUT), lambda i: (i, 0)),
        out_shape=jax.ShapeDtypeStruct((N_NODES, D_OUT), jnp.float32),
    )(x, w)


def _cntsum_body(c_ref, o_ref):
    o_ref[...] = jnp.sum(c_ref[...], axis=(0, 1)).astype(jnp.float32)


def _cntsum(cnt4):
    return pl.pallas_call(
        _cntsum_body,
        out_shape=jax.ShapeDtypeStruct((HR, HC), jnp.float32),
    )(cnt4)


_EW_BLOCK = 2000


def _scale_body(c_ref, x_ref, o_ref):
    deg = 2.0 + c_ref[...]
    o_ref[...] = x_ref[...] * lax.rsqrt(deg)


def _scale(cnt_col, xw):
    return pl.pallas_call(
        _scale_body,
        grid=(N_NODES // _EW_BLOCK,),
        in_specs=[pl.BlockSpec((_EW_BLOCK, 1), lambda i: (i, 0)),
                  pl.BlockSpec((_EW_BLOCK, D_OUT), lambda i: (i, 0))],
        out_specs=pl.BlockSpec((_EW_BLOCK, D_OUT), lambda i: (i, 0)),
        out_shape=jax.ShapeDtypeStruct((N_NODES, D_OUT), jnp.float32),
    )(cnt_col, xw)


def _final_body(a_ref, c_ref, x_ref, o_ref):
    deg = 2.0 + c_ref[...]
    dinv = lax.rsqrt(deg)
    tot = a_ref[0] + a_ref[1]
    o_ref[...] = jnp.maximum(dinv * tot + (2.0 / deg) * x_ref[...], 0.0)


def _final(acc, cnt_col, xw):
    return pl.pallas_call(
        _final_body,
        grid=(N_NODES // _EW_BLOCK,),
        in_specs=[pl.BlockSpec((NC, _EW_BLOCK, D_OUT), lambda i: (0, i, 0)),
                  pl.BlockSpec((_EW_BLOCK, 1), lambda i: (i, 0)),
                  pl.BlockSpec((_EW_BLOCK, D_OUT), lambda i: (i, 0))],
        out_specs=pl.BlockSpec((_EW_BLOCK, D_OUT), lambda i: (i, 0)),
        out_shape=jax.ShapeDtypeStruct((N_NODES, D_OUT), jnp.float32),
    )(acc, cnt_col, xw)


def kernel(x, edge_index, W):
    rows = edge_index[0].reshape(NC, NS, NB, SB, EDGE_K)
    cols = edge_index[1].reshape(NC, NS, NB, SB, EDGE_K)
    zeros128 = jnp.zeros((ROWS_PER_TILE, D_OUT), jnp.float32)

    cnt4 = _hist(rows)
    cntf = _cntsum(cnt4)
    cnt_col = cntf.reshape(N_PAD)[:N_NODES].reshape(N_NODES, 1)
    xw = _matmul(x, W)
    z = _scale(cnt_col, xw)
    acc = _agg(z, rows, cols, zeros128)
    return _final(acc, cnt_col, xw)


# speedup vs baseline: 42.0468x; 1.1083x over previous
import jax
import jax.numpy as jnp
from jax import lax
from jax.experimental import pallas as pl
from jax.experimental.pallas import tpu as pltpu
from jax.experimental.pallas import tpu_sc as plsc

N_NODES = 10000
D_IN = 128
D_OUT = 128
N_EDGES = 320000

NC, NS = 2, 16
EDGES_PER_TILE = N_EDGES // (NC * NS)
EDGE_K = 80
SB = 25
NB = EDGES_PER_TILE // (EDGE_K * SB)
N_PAD = 10240
ROWS_PER_TILE = N_PAD // NS
HR, HC = N_PAD // 128, 128

_MESH = plsc.VectorSubcoreMesh(core_axis_name="c", subcore_axis_name="s")


def _hist_body(rows_hbm, cnt_hbm, idx_v, hist_v):
    c = lax.axis_index("c")
    s = lax.axis_index("s")

    @pl.loop(0, HR)
    def _(r):
        for l in range(HC // 16):
            hist_v[r, pl.ds(l * 16, 16)] = jnp.zeros((16,), jnp.int32)

    @pl.loop(0, NB)
    def _(b):
        pltpu.sync_copy(rows_hbm.at[c, s, b], idx_v)

        @pl.loop(0, SB)
        def _(t):
            for l in range(EDGE_K // 16):
                v = idx_v[t, pl.ds(l * 16, 16)]
                cnts, last = plsc.scan_count(v)
                hi = lax.shift_right_logical(v, 7)
                lo = lax.bitwise_and(v, 127)
                plsc.addupdate_scatter(hist_v, [hi, lo], cnts, mask=last)

    pltpu.sync_copy(hist_v, cnt_hbm.at[c, s])


_hist = pl.kernel(
    _hist_body,
    out_type=jax.ShapeDtypeStruct((NC, NS, HR, HC), jnp.int32),
    mesh=_MESH,
    scratch_types=[
        pltpu.VMEM((SB, EDGE_K), jnp.int32),
        pltpu.VMEM((HR, HC), jnp.int32),
    ],
    compiler_params=pltpu.CompilerParams(needs_layout_passes=False),
)


def _agg_body(z_hbm, rows_hbm, cols_hbm, zeros_hbm, out_hbm,
              ridx_v, cidx_v, upd0, upd1, upd2, acc_sp,
              gsem0, gsem1, gsem2, ssem0, ssem1, ssem2):
    c = lax.axis_index("c")
    s = lax.axis_index("s")
    base = s * ROWS_PER_TILE
    pltpu.sync_copy(zeros_hbm, acc_sp.at[pl.ds(base, ROWS_PER_TILE)])
    plsc.subcore_barrier()

    bufs = (upd0, upd1, upd2)
    gsems = (gsem0, gsem1, gsem2)
    ssems = (ssem0, ssem1, ssem2)
    NBUF = 3

    @pl.loop(0, NB)
    def _(b):
        pltpu.sync_copy(rows_hbm.at[c, s, b], ridx_v)
        pltpu.sync_copy(cols_hbm.at[c, s, b], cidx_v)
        gd = [None] * SB
        sd = [None] * SB
        for t in range(NBUF):
            gd[t] = pltpu.async_copy(z_hbm.at[cidx_v.at[t]], bufs[t],
                                     gsems[t])
        for t in range(SB):
            p = t % NBUF
            gd[t].wait()
            sd[t] = pltpu.async_copy(bufs[p], acc_sp.at[ridx_v.at[t]],
                                     ssems[p], add=True)
            if t + NBUF < SB:
                sd[t].wait()
                gd[t + NBUF] = pltpu.async_copy(z_hbm.at[cidx_v.at[t + NBUF]],
                                                bufs[p], gsems[p])
        for t in range(SB - NBUF, SB):
            sd[t].wait()

    plsc.subcore_barrier()
    pltpu.sync_copy(acc_sp.at[pl.ds(base, ROWS_PER_TILE)],
                    out_hbm.at[c, pl.ds(base, ROWS_PER_TILE)])


_agg = pl.kernel(
    _agg_body,
    out_type=jax.ShapeDtypeStruct((NC, N_PAD, D_OUT), jnp.float32),
    mesh=_MESH,
    scratch_types=[
        pltpu.VMEM((SB, EDGE_K), jnp.int32),
        pltpu.VMEM((SB, EDGE_K), jnp.int32),
        pltpu.VMEM((EDGE_K, D_OUT), jnp.float32),
        pltpu.VMEM((EDGE_K, D_OUT), jnp.float32),
        pltpu.VMEM((EDGE_K, D_OUT), jnp.float32),
        pltpu.VMEM_SHARED((N_PAD, D_OUT), jnp.float32),
        pltpu.SemaphoreType.DMA,
        pltpu.SemaphoreType.DMA,
        pltpu.SemaphoreType.DMA,
        pltpu.SemaphoreType.DMA,
        pltpu.SemaphoreType.DMA,
        pltpu.SemaphoreType.DMA,
    ],
)


_MM_BLOCK = 2000


def _mm_body(x_ref, w_ref, o_ref):
    o_ref[...] = jnp.dot(x_ref[...], w_ref[...],
                         preferred_element_type=jnp.float32)


def _matmul(x, w):
    return pl.pallas_call(
        _mm_body,
        grid=(N_NODES // _MM_BLOCK,),
        in_specs=[pl.BlockSpec((_MM_BLOCK, D_IN), lambda i: (i, 0)),
                  pl.BlockSpec((D_IN, D_OUT), lambda i: (0, 0))],
        out_specs=pl.BlockSpec((_MM_BLOCK, D_OUT), lambda i: (i, 0)),
        out_shape=jax.ShapeDtypeStruct((N_NODES, D_OUT), jnp.float32),
    )(x, w)


def _cntsum_body(c_ref, o_ref):
    o_ref[...] = jnp.sum(c_ref[...], axis=(0, 1)).astype(jnp.float32)


def _cntsum(cnt4):
    return pl.pallas_call(
        _cntsum_body,
        out_shape=jax.ShapeDtypeStruct((HR, HC), jnp.float32),
    )(cnt4)


_EW_BLOCK = 2000


def _scale_body(c_ref, x_ref, o_ref):
    deg = 2.0 + c_ref[...]
    o_ref[...] = x_ref[...] * lax.rsqrt(deg)


def _scale(cnt_col, xw):
    return pl.pallas_call(
        _scale_body,
        grid=(N_NODES // _EW_BLOCK,),
        in_specs=[pl.BlockSpec((_EW_BLOCK, 1), lambda i: (i, 0)),
                  pl.BlockSpec((_EW_BLOCK, D_OUT), lambda i: (i, 0))],
        out_specs=pl.BlockSpec((_EW_BLOCK, D_OUT), lambda i: (i, 0)),
        out_shape=jax.ShapeDtypeStruct((N_NODES, D_OUT), jnp.float32),
    )(cnt_col, xw)


def _final_body(a_ref, c_ref, x_ref, o_ref):
    deg = 2.0 + c_ref[...]
    dinv = lax.rsqrt(deg)
    tot = a_ref[0] + a_ref[1]
    o_ref[...] = jnp.maximum(dinv * tot + (2.0 / deg) * x_ref[...], 0.0)


def _final(acc, cnt_col, xw):
    return pl.pallas_call(
        _final_body,
        grid=(N_NODES // _EW_BLOCK,),
        in_specs=[pl.BlockSpec((NC, _EW_BLOCK, D_OUT), lambda i: (0, i, 0)),
                  pl.BlockSpec((_EW_BLOCK, 1), lambda i: (i, 0)),
                  pl.BlockSpec((_EW_BLOCK, D_OUT), lambda i: (i, 0))],
        out_specs=pl.BlockSpec((_EW_BLOCK, D_OUT), lambda i: (i, 0)),
        out_shape=jax.ShapeDtypeStruct((N_NODES, D_OUT), jnp.float32),
    )(acc, cnt_col, xw)


def kernel(x, edge_index, W):
    rows = edge_index[0].reshape(NC, NS, NB, SB, EDGE_K)
    cols = edge_index[1].reshape(NC, NS, NB, SB, EDGE_K)
    zeros128 = jnp.zeros((ROWS_PER_TILE, D_OUT), jnp.float32)

    cnt4 = _hist(rows)
    cntf = _cntsum(cnt4)
    cnt_col = cntf.reshape(N_PAD)[:N_NODES].reshape(N_NODES, 1)
    xw = _matmul(x, W)
    z = _scale(cnt_col, xw)
    acc = _agg(z, rows, cols, zeros128)
    return _final(acc, cnt_col, xw)
